# baseline (speedup 1.0000x reference)
import jax
import jax.numpy as jnp
from jax.experimental import pallas as pl
from jax.experimental.pallas import tpu as pltpu

_N_ROWS = 16384
_N_COLS = 512
_BLK = 4096
_GRID = _N_ROWS // _BLK
_TOTAL = float(_N_ROWS * _N_COLS)


def _loss_body(x_ref, t_ref, out_ref, acc_ref):
    i = pl.program_id(0)

    @pl.when(i == 0)
    def _init():
        acc_ref[0] = 0.0
        acc_ref[1] = 0.0
        acc_ref[2] = 0.0

    x = x_ref[...]
    t = t_ref[...]
    y = x * (1.0 - 2.0 * t)
    bce = jnp.maximum(y, 0.0) + jnp.log(1.0 + jnp.exp(-jnp.abs(y)))
    acc_ref[0] += jnp.sum(bce * t)
    acc_ref[1] += jnp.sum(bce)
    acc_ref[2] += jnp.sum(t)

    @pl.when(i == _GRID - 1)
    def _finalize():
        pos_sum = acc_ref[0]
        all_sum = acc_ref[1]
        pos_cnt = acc_ref[2]
        neg_sum = all_sum - pos_sum
        pos_loss = 0.5 * pos_sum / jnp.maximum(pos_cnt, 1.0)
        neg_loss = 0.5 * neg_sum / jnp.maximum(_TOTAL - pos_cnt, 1.0)
        out_ref[0] = pos_loss + neg_loss
        out_ref[1] = pos_loss
        out_ref[2] = neg_loss


def kernel(font_output_data, font_target_data):
    out = pl.pallas_call(
        _loss_body,
        grid=(_GRID,),
        in_specs=[
            pl.BlockSpec((_BLK, _N_COLS), lambda i: (i, 0)),
            pl.BlockSpec((_BLK, _N_COLS), lambda i: (i, 0)),
        ],
        out_specs=pl.BlockSpec(memory_space=pltpu.SMEM),
        out_shape=jax.ShapeDtypeStruct((3,), jnp.float32),
        scratch_shapes=[pltpu.SMEM((3,), jnp.float32)],
    )(font_output_data, font_target_data)
    return (out[0], out[1], out[2])


# speedup vs baseline: 1.0028x; 1.0028x over previous
import jax
import jax.numpy as jnp
from jax.experimental import pallas as pl
from jax.experimental.pallas import tpu as pltpu

_N_ROWS = 16384
_N_COLS = 512
_BLK = 2048
_GRID = _N_ROWS // _BLK
_TOTAL = float(_N_ROWS * _N_COLS)


def _loss_body(x_ref, t_ref, out_ref, acc_ref):
    i = pl.program_id(0)

    @pl.when(i == 0)
    def _init():
        acc_ref[0] = 0.0
        acc_ref[1] = 0.0
        acc_ref[2] = 0.0

    x = x_ref[...]
    t = t_ref[...]
    neg_log2e = jnp.float32(-1.4426950408889634)
    ln2 = jnp.float32(0.6931471805599453)
    bce = (jnp.maximum(x, 0.0) - t * x) + ln2 * jnp.log2(
        1.0 + jnp.exp2(neg_log2e * jnp.abs(x))
    )
    acc_ref[0] += jnp.sum(bce * t)
    acc_ref[1] += jnp.sum(bce)
    acc_ref[2] += jnp.sum(t)

    @pl.when(i == _GRID - 1)
    def _finalize():
        pos_sum = acc_ref[0]
        all_sum = acc_ref[1]
        pos_cnt = acc_ref[2]
        neg_sum = all_sum - pos_sum
        pos_loss = 0.5 * pos_sum / jnp.maximum(pos_cnt, 1.0)
        neg_loss = 0.5 * neg_sum / jnp.maximum(_TOTAL - pos_cnt, 1.0)
        out_ref[0] = pos_loss + neg_loss
        out_ref[1] = pos_loss
        out_ref[2] = neg_loss


def kernel(font_output_data, font_target_data):
    out = pl.pallas_call(
        _loss_body,
        grid=(_GRID,),
        in_specs=[
            pl.BlockSpec((_BLK, _N_COLS), lambda i: (i, 0)),
            pl.BlockSpec((_BLK, _N_COLS), lambda i: (i, 0)),
        ],
        out_specs=pl.BlockSpec(memory_space=pltpu.SMEM),
        out_shape=jax.ShapeDtypeStruct((3,), jnp.float32),
        scratch_shapes=[pltpu.SMEM((3,), jnp.float32)],
    )(font_output_data, font_target_data)
    return (out[0], out[1], out[2])


# speedup vs baseline: 1.2470x; 1.2436x over previous
import jax
import jax.numpy as jnp
from jax.experimental import pallas as pl
from jax.experimental.pallas import tpu as pltpu

_N_ROWS = 16384
_N_COLS = 512
_BLK = 2048
_GRID = _N_ROWS // _BLK
_TOTAL = float(_N_ROWS * _N_COLS)


def _loss_body(x_ref, t_ref, out_ref, acc_ref):
    i = pl.program_id(0)

    @pl.when(i == 0)
    def _init():
        acc_ref[...] = jnp.zeros_like(acc_ref)

    x = x_ref[...]
    t = t_ref[...]
    y = x * (1.0 - 2.0 * t)
    bce = jnp.maximum(y, 0.0) + jnp.log(1.0 + jnp.exp(-jnp.abs(y)))
    ones = jnp.ones((8, _BLK), jnp.float32)
    acc_ref[0] += jnp.dot(ones, bce * t, preferred_element_type=jnp.float32)
    acc_ref[1] += jnp.dot(ones, bce, preferred_element_type=jnp.float32)
    acc_ref[2] += jnp.dot(ones, t, preferred_element_type=jnp.float32)

    @pl.when(i == _GRID - 1)
    def _finalize():
        pos_sum = jnp.sum(acc_ref[0]) * 0.125
        all_sum = jnp.sum(acc_ref[1]) * 0.125
        pos_cnt = jnp.sum(acc_ref[2]) * 0.125
        neg_sum = all_sum - pos_sum
        pos_loss = 0.5 * pos_sum / jnp.maximum(pos_cnt, 1.0)
        neg_loss = 0.5 * neg_sum / jnp.maximum(_TOTAL - pos_cnt, 1.0)
        out_ref[0, 0] = pos_loss + neg_loss
        out_ref[0, 1] = pos_loss
        out_ref[0, 2] = neg_loss


def kernel(font_output_data, font_target_data):
    out = pl.pallas_call(
        _loss_body,
        grid=(_GRID,),
        in_specs=[
            pl.BlockSpec((_BLK, _N_COLS), lambda i: (i, 0)),
            pl.BlockSpec((_BLK, _N_COLS), lambda i: (i, 0)),
        ],
        out_specs=pl.BlockSpec(memory_space=pltpu.SMEM),
        out_shape=jax.ShapeDtypeStruct((1, 3), jnp.float32),
        scratch_shapes=[pltpu.VMEM((3, 8, _N_COLS), jnp.float32)],
    )(font_output_data, font_target_data)
    return (out[0, 0], out[0, 1], out[0, 2])


# speedup vs baseline: 1.3017x; 1.0438x over previous
import jax
import jax.numpy as jnp
from jax.experimental import pallas as pl
from jax.experimental.pallas import tpu as pltpu

_N_ROWS = 16384
_N_COLS = 512
_BLK = 2048
_GRID = _N_ROWS // _BLK
_TOTAL = float(_N_ROWS * _N_COLS)


def _loss_body(x_ref, t_ref, out_ref, acc_ref):
    i = pl.program_id(0)

    @pl.when(i == 0)
    def _init():
        acc_ref[...] = jnp.zeros_like(acc_ref)

    x = x_ref[...]
    t = t_ref[...]
    xi = x.view(jnp.int32)
    y = (xi ^ (t.view(jnp.int32) << 8)).view(jnp.float32)
    neg_abs = (xi | jnp.int32(-2147483648)).view(jnp.float32)
    bce = jnp.maximum(y, 0.0) + jnp.log(1.0 + jnp.exp(neg_abs))
    ones = jnp.ones((8, _BLK), jnp.float32)
    acc_ref[0] += jnp.dot(ones, bce * t, preferred_element_type=jnp.float32)
    acc_ref[1] += jnp.dot(ones, bce, preferred_element_type=jnp.float32)
    acc_ref[2] += jnp.dot(ones, t, preferred_element_type=jnp.float32)

    @pl.when(i == _GRID - 1)
    def _finalize():
        pos_sum = jnp.sum(acc_ref[0]) * 0.125
        all_sum = jnp.sum(acc_ref[1]) * 0.125
        pos_cnt = jnp.sum(acc_ref[2]) * 0.125
        neg_sum = all_sum - pos_sum
        pos_loss = 0.5 * pos_sum / jnp.maximum(pos_cnt, 1.0)
        neg_loss = 0.5 * neg_sum / jnp.maximum(_TOTAL - pos_cnt, 1.0)
        out_ref[0, 0] = pos_loss + neg_loss
        out_ref[0, 1] = pos_loss
        out_ref[0, 2] = neg_loss


def kernel(font_output_data, font_target_data):
    out = pl.pallas_call(
        _loss_body,
        grid=(_GRID,),
        in_specs=[
            pl.BlockSpec((_BLK, _N_COLS), lambda i: (i, 0)),
            pl.BlockSpec((_BLK, _N_COLS), lambda i: (i, 0)),
        ],
        out_specs=pl.BlockSpec(memory_space=pltpu.SMEM),
        out_shape=jax.ShapeDtypeStruct((1, 3), jnp.float32),
        scratch_shapes=[pltpu.VMEM((3, 8, _N_COLS), jnp.float32)],
    )(font_output_data, font_target_data)
    return (out[0, 0], out[0, 1], out[0, 2])
